# baseline (speedup 1.0000x reference)
import functools

import jax
import jax.numpy as jnp
from jax import lax
from jax.experimental import pallas as pl
from jax.experimental.pallas import tpu as pltpu
from jax.experimental.pallas import tpu_sc as plsc

_B, _N, _M, _C = 4, 1024, 4096, 256
_K = 32
_CMID, _COUT = 256, 512
_TN = 128
_NW = 32
_CH = 128


def _prep_body(feat_ref, sxyz_ref, w1f_ref, w1dp_ref, b1_ref, out_ref):
    f = feat_ref[0]
    s = sxyz_ref[0]
    h = lax.dot_general(f, w1f_ref[...], (((0,), (1,)), ((), ())),
                        preferred_element_type=jnp.float32)
    h = h + lax.dot_general(s, w1dp_ref[...], (((1,), (1,)), ((), ())),
                            preferred_element_type=jnp.float32)
    out_ref[0] = h + b1_ref[...]


def _make_table(features, support_xyz, w1f, w1dp, b1):
    return pl.pallas_call(
        _prep_body,
        grid=(1,),
        in_specs=[
            pl.BlockSpec((1, _C, _M), lambda b: (b, 0, 0)),
            pl.BlockSpec((1, _M, 3), lambda b: (b, 0, 0)),
            pl.BlockSpec((_CMID, _C), lambda b: (0, 0)),
            pl.BlockSpec((_CMID, 3), lambda b: (0, 0)),
            pl.BlockSpec((1, _CMID), lambda b: (0, 0)),
        ],
        out_specs=pl.BlockSpec((1, _M, _CMID), lambda b: (b, 0, 0)),
        out_shape=jax.ShapeDtypeStruct((1, _M, _CMID), jnp.float32),
    )(features, support_xyz, w1f, w1dp, b1)


def _topk_body(q_ref, st_ref, idx_ref):
    q = q_ref[0]
    s = st_ref[0]
    qn = jnp.sum(q * q, axis=1, keepdims=True)
    sn = jnp.sum(s * s, axis=0, keepdims=True)
    dot = lax.dot_general(q, s, (((1,), (0,)), ((), ())),
                          preferred_element_type=jnp.float32)
    d = jnp.maximum(qn + sn - 2.0 * dot, 0.0)
    bits = lax.bitcast_convert_type(d, jnp.int32)
    iota = lax.broadcasted_iota(jnp.int32, (_TN, _M), 1)
    key = jnp.bitwise_and(bits, jnp.int32(-4096)) | iota
    m = jnp.min(key, axis=1, keepdims=True)
    cols = [jnp.bitwise_and(m, 4095)]
    for _ in range(_K - 1):
        key = jnp.where(key == m, jnp.int32(0x7FFFFFFF), key)
        m = jnp.min(key, axis=1, keepdims=True)
        cols.append(jnp.bitwise_and(m, 4095))
    idx_ref[0] = jnp.concatenate(cols, axis=1)


def _topk(query_xyz, support_t):
    return pl.pallas_call(
        _topk_body,
        grid=(1, _N // _TN),
        in_specs=[
            pl.BlockSpec((1, _TN, 3), lambda b, t: (b, t, 0)),
            pl.BlockSpec((1, 3, _M), lambda b, t: (b, 0, 0)),
        ],
        out_specs=pl.BlockSpec((1, _TN, _K), lambda b, t: (b, t, 0)),
        out_shape=jax.ShapeDtypeStruct((1, _N, _K), jnp.int32),
    )(query_xyz, support_t)


_BT = _K * _N
_BPW = _BT // _NW


def _sc_gather_body(table_hbm, idx_hbm, out_hbm, idx_v, rows_v, sem):
    wid = lax.axis_index("s") * 2 + lax.axis_index("c")
    base = wid * _BPW

    def chunk(c, carry):
        off = base + c * _CH
        pltpu.sync_copy(idx_hbm.at[pl.ds(off, _CH)], idx_v)
        pltpu.async_copy(table_hbm.at[idx_v], rows_v, sem).wait()
        pltpu.sync_copy(rows_v, out_hbm.at[pl.ds(off, _CH)])
        return carry

    lax.fori_loop(0, _BPW // _CH, chunk, 0)


@functools.cache
def _sc_gather_fn():
    return pl.kernel(
        _sc_gather_body,
        out_type=jax.ShapeDtypeStruct((_BT, _CMID), jnp.float32),
        mesh=plsc.VectorSubcoreMesh(core_axis_name="c", subcore_axis_name="s"),
        scratch_types=[
            pltpu.VMEM((_CH,), jnp.int32),
            pltpu.VMEM((_CH, _CMID), jnp.float32),
            pltpu.SemaphoreType.DMA,
        ],
    )


def _mlp_body(g_ref, q_ref, w1dp_ref, w2_ref, b2_ref, out_ref):
    q = q_ref[0]
    corr = lax.dot_general(q, w1dp_ref[...], (((1,), (1,)), ((), ())),
                           preferred_element_type=jnp.float32)
    w2 = w2_ref[...]
    acc = jnp.full((_COUT, _TN), -jnp.inf, dtype=jnp.float32)
    for k in range(_K):
        h1 = jnp.maximum(g_ref[0, k] - corr, 0.0)
        h2 = lax.dot_general(w2, h1.astype(jnp.bfloat16),
                             (((1,), (1,)), ((), ())),
                             preferred_element_type=jnp.float32)
        acc = jnp.maximum(acc, h2)
    out_ref[0] = jnp.maximum(acc + b2_ref[...], 0.0)


def _mlp(gathered, query_xyz, w1dp, w2, b2):
    return pl.pallas_call(
        _mlp_body,
        grid=(1, _N // _TN),
        in_specs=[
            pl.BlockSpec((1, _K, _TN, _CMID), lambda b, t: (b, 0, t, 0)),
            pl.BlockSpec((1, _TN, 3), lambda b, t: (b, t, 0)),
            pl.BlockSpec((_CMID, 3), lambda b, t: (0, 0)),
            pl.BlockSpec((_COUT, _CMID), lambda b, t: (0, 0)),
            pl.BlockSpec((_COUT, 1), lambda b, t: (0, 0)),
        ],
        out_specs=pl.BlockSpec((1, _COUT, _TN), lambda b, t: (b, 0, t)),
        out_shape=jax.ShapeDtypeStruct((1, _COUT, _N), jnp.float32),
    )(gathered, query_xyz, w1dp, w2, b2)


@jax.jit
def kernel(query_xyz, support_xyz, features, W1, b1, W2, b2):
    w1dp = W1[:, :3]
    w1f = W1[:, 3:]
    support_t = jnp.swapaxes(support_xyz, 1, 2)
    b1r = b1.reshape(1, _CMID)
    b2r = b2.reshape(_COUT, 1)
    w2b = W2.astype(jnp.bfloat16)
    gather = _sc_gather_fn()

    outs = []
    for b in range(_B):
        qb = lax.slice_in_dim(query_xyz, b, b + 1, axis=0)
        table = _make_table(lax.slice_in_dim(features, b, b + 1, axis=0),
                            lax.slice_in_dim(support_xyz, b, b + 1, axis=0),
                            w1f, w1dp, b1r)
        idx = _topk(qb, lax.slice_in_dim(support_t, b, b + 1, axis=0))
        flat_idx = jnp.swapaxes(idx, 1, 2).reshape(_BT)
        g = gather(table.reshape(_M, _CMID), flat_idx)
        outs.append(_mlp(g.reshape(1, _K, _N, _CMID), qb, w1dp, w2b, b2r))
    return jnp.concatenate(outs, axis=0)


# speedup vs baseline: 1.1358x; 1.1358x over previous
import functools

import jax
import jax.numpy as jnp
from jax import lax
from jax.experimental import pallas as pl
from jax.experimental.pallas import tpu as pltpu
from jax.experimental.pallas import tpu_sc as plsc

_B, _N, _M, _C = 4, 1024, 4096, 256
_K = 32
_CMID, _COUT = 256, 512
_TN = 128
_NW = 32
_CH = 128


def _prep_body(feat_ref, sxyz_ref, w1f_ref, w1dp_ref, b1_ref, out_ref):
    f = feat_ref[0]
    s = sxyz_ref[0]
    h = lax.dot_general(f, w1f_ref[...], (((0,), (1,)), ((), ())),
                        preferred_element_type=jnp.float32)
    h = h + lax.dot_general(s, w1dp_ref[...], (((1,), (1,)), ((), ())),
                            preferred_element_type=jnp.float32)
    out_ref[0] = h + b1_ref[...]


def _make_table(features, support_xyz, w1f, w1dp, b1):
    return pl.pallas_call(
        _prep_body,
        grid=(1,),
        in_specs=[
            pl.BlockSpec((1, _C, _M), lambda b: (b, 0, 0)),
            pl.BlockSpec((1, _M, 3), lambda b: (b, 0, 0)),
            pl.BlockSpec((_CMID, _C), lambda b: (0, 0)),
            pl.BlockSpec((_CMID, 3), lambda b: (0, 0)),
            pl.BlockSpec((1, _CMID), lambda b: (0, 0)),
        ],
        out_specs=pl.BlockSpec((1, _M, _CMID), lambda b: (b, 0, 0)),
        out_shape=jax.ShapeDtypeStruct((1, _M, _CMID), jnp.float32),
    )(features, support_xyz, w1f, w1dp, b1)


def _topk_body(q_ref, st_ref, idx_ref):
    q = q_ref[0]
    s = st_ref[0]
    qn = jnp.sum(q * q, axis=1, keepdims=True)
    sn = jnp.sum(s * s, axis=0, keepdims=True)
    dot = lax.dot_general(q, s, (((1,), (0,)), ((), ())),
                          preferred_element_type=jnp.float32)
    d = jnp.maximum(qn + sn - 2.0 * dot, 0.0)
    bits = lax.bitcast_convert_type(d, jnp.int32)
    iota = lax.broadcasted_iota(jnp.int32, (_TN, _M), 1)
    key = jnp.bitwise_and(bits, jnp.int32(-4096)) | iota
    m = jnp.min(key, axis=1, keepdims=True)
    cols = [m]
    for _ in range(_K - 1):
        madj = m + jnp.int32(-2147483647)
        m = madj + jnp.min(key - madj, axis=1, keepdims=True)
        cols.append(m)
    idx = jnp.concatenate(cols, axis=1)
    idx_ref[0] = jnp.bitwise_and(idx, 4095)


def _topk(query_xyz, support_t):
    return pl.pallas_call(
        _topk_body,
        grid=(1, _N // _TN),
        in_specs=[
            pl.BlockSpec((1, _TN, 3), lambda b, t: (b, t, 0)),
            pl.BlockSpec((1, 3, _M), lambda b, t: (b, 0, 0)),
        ],
        out_specs=pl.BlockSpec((1, _TN, _K), lambda b, t: (b, t, 0)),
        out_shape=jax.ShapeDtypeStruct((1, _N, _K), jnp.int32),
    )(query_xyz, support_t)


_BT = _K * _N
_BPW = _BT // _NW


def _sc_gather_body(table_hbm, idx_hbm, out_hbm, idx_v, rows_v, sem):
    wid = lax.axis_index("s") * 2 + lax.axis_index("c")
    base = wid * _BPW

    def chunk(c, carry):
        off = base + c * _CH
        pltpu.sync_copy(idx_hbm.at[pl.ds(off, _CH)], idx_v)
        pltpu.async_copy(table_hbm.at[idx_v], rows_v, sem).wait()
        pltpu.sync_copy(rows_v, out_hbm.at[pl.ds(off, _CH)])
        return carry

    lax.fori_loop(0, _BPW // _CH, chunk, 0)


@functools.cache
def _sc_gather_fn():
    return pl.kernel(
        _sc_gather_body,
        out_type=jax.ShapeDtypeStruct((_BT, _CMID), jnp.float32),
        mesh=plsc.VectorSubcoreMesh(core_axis_name="c", subcore_axis_name="s"),
        scratch_types=[
            pltpu.VMEM((_CH,), jnp.int32),
            pltpu.VMEM((_CH, _CMID), jnp.float32),
            pltpu.SemaphoreType.DMA,
        ],
    )


def _mlp_body(g_ref, q_ref, w1dp_ref, w2_ref, b2_ref, out_ref):
    q = q_ref[0]
    corr = lax.dot_general(q, w1dp_ref[...], (((1,), (1,)), ((), ())),
                           preferred_element_type=jnp.float32)
    w2 = w2_ref[...]
    acc = jnp.full((_COUT, _TN), -jnp.inf, dtype=jnp.float32)
    for k in range(_K):
        h1 = jnp.maximum(g_ref[0, k] - corr, 0.0)
        h2 = lax.dot_general(w2, h1, (((1,), (1,)), ((), ())),
                             preferred_element_type=jnp.float32)
        acc = jnp.maximum(acc, h2)
    out_ref[0] = jnp.maximum(acc + b2_ref[...], 0.0)


def _mlp(gathered, query_xyz, w1dp, w2, b2):
    return pl.pallas_call(
        _mlp_body,
        grid=(1, _N // _TN),
        in_specs=[
            pl.BlockSpec((1, _K, _TN, _CMID), lambda b, t: (b, 0, t, 0)),
            pl.BlockSpec((1, _TN, 3), lambda b, t: (b, t, 0)),
            pl.BlockSpec((_CMID, 3), lambda b, t: (0, 0)),
            pl.BlockSpec((_COUT, _CMID), lambda b, t: (0, 0)),
            pl.BlockSpec((_COUT, 1), lambda b, t: (0, 0)),
        ],
        out_specs=pl.BlockSpec((1, _COUT, _TN), lambda b, t: (b, 0, t)),
        out_shape=jax.ShapeDtypeStruct((1, _COUT, _N), jnp.float32),
    )(gathered, query_xyz, w1dp, w2, b2)


@jax.jit
def kernel(query_xyz, support_xyz, features, W1, b1, W2, b2):
    w1dp = W1[:, :3]
    w1f = W1[:, 3:]
    support_t = jnp.swapaxes(support_xyz, 1, 2)
    b1r = b1.reshape(1, _CMID)
    b2r = b2.reshape(_COUT, 1)
    gather = _sc_gather_fn()

    outs = []
    for b in range(_B):
        qb = lax.slice_in_dim(query_xyz, b, b + 1, axis=0)
        table = _make_table(lax.slice_in_dim(features, b, b + 1, axis=0),
                            lax.slice_in_dim(support_xyz, b, b + 1, axis=0),
                            w1f, w1dp, b1r)
        idx = _topk(qb, lax.slice_in_dim(support_t, b, b + 1, axis=0))
        flat_idx = jnp.swapaxes(idx, 1, 2).reshape(_BT)
        g = gather(table.reshape(_M, _CMID), flat_idx)
        outs.append(_mlp(g.reshape(1, _K, _N, _CMID), qb, w1dp, W2, b2r))
    return jnp.concatenate(outs, axis=0)
